# baseline (speedup 1.0000x reference)
import functools
import math

import jax
import jax.numpy as jnp
from jax import lax
from jax.experimental import pallas as pl
from jax.experimental.pallas import tpu as pltpu
from jax.experimental.pallas import tpu_sc as plsc

B, S, H = 8, 512, 12
REL_POS_BINS, MAX_REL_POS = 32, 128
REL_2D_POS_BINS, MAX_REL_2D_POS = 64, 256

POS_RANGE = 512
COORD_RANGE = 1000
LUT1_PAD = 1024
LUT2_PAD = 2048

NPLANES = B * H
NWORKERS = 32
PW = NPLANES // NWORKERS
ROWS = 16
NCHUNK = S // ROWS
LANE = 16


def _bucket_ids(delta, num_buckets, max_distance):
    nb = num_buckets // 2
    ret = (delta > 0).astype(delta.dtype) * nb
    n = jnp.abs(delta)
    max_exact = nb // 2
    is_small = n < max_exact
    val_if_large = max_exact + (
        jnp.log(n.astype(jnp.float32) / max_exact)
        / math.log(max_distance / max_exact)
        * (nb - max_exact)
    ).astype(delta.dtype)
    val_if_large = jnp.minimum(val_if_large, jnp.full_like(val_if_large, nb - 1))
    return ret + jnp.where(is_small, n, val_if_large)


def _sc_bias_fn():
    mesh = plsc.VectorSubcoreMesh(
        core_axis_name="c", subcore_axis_name="s", num_cores=2, num_subcores=16
    )

    @functools.partial(
        pl.kernel,
        out_type=jax.ShapeDtypeStruct((NPLANES, S, S), jnp.float32),
        mesh=mesh,
        compiler_params=pltpu.CompilerParams(needs_layout_passes=False),
        scratch_types=[
            pltpu.VMEM((S,), jnp.int32),
            pltpu.VMEM((S,), jnp.int32),
            pltpu.VMEM((S,), jnp.int32),
            pltpu.VMEM((S + LANE,), jnp.int32),
            pltpu.VMEM((S + LANE,), jnp.int32),
            pltpu.VMEM((S + LANE,), jnp.int32),
            pltpu.VMEM((LUT1_PAD,), jnp.int32),
            pltpu.VMEM((LUT2_PAD,), jnp.int32),
            pltpu.VMEM((REL_POS_BINS,), jnp.float32),
            pltpu.VMEM((REL_2D_POS_BINS,), jnp.float32),
            pltpu.VMEM((REL_2D_POS_BINS,), jnp.float32),
            pltpu.VMEM((PW * LUT1_PAD,), jnp.float32),
            pltpu.VMEM((PW * LUT2_PAD,), jnp.float32),
            pltpu.VMEM((PW * LUT2_PAD,), jnp.float32),
            pltpu.VMEM((2 * PW, ROWS, S), jnp.float32),
            pltpu.SemaphoreType.DMA,
            pltpu.SemaphoreType.DMA,
        ],
    )
    def sc_bias(pos_hbm, x_hbm, y_hbm, lut1_hbm, lut2_hbm,
                w1_hbm, wx_hbm, wy_hbm, out_hbm,
                pj_v, xj_v, yj_v, sp_v, sx_v, sy_v,
                lut1_v, lut2_v, w1_v, wx_v, wy_v,
                t1_v, tx_v, ty_v, buf_v, semA, semB):
        wid = lax.axis_index("s") * 2 + lax.axis_index("c")
        cplane0 = wid * PW
        b = cplane0 // H
        h0 = cplane0 - b * H

        pltpu.sync_copy(lut1_hbm, lut1_v)
        pltpu.sync_copy(lut2_hbm, lut2_v)
        pltpu.sync_copy(pos_hbm.at[b], pj_v)
        pltpu.sync_copy(x_hbm.at[b], xj_v)
        pltpu.sync_copy(y_hbm.at[b], yj_v)

        def shift_body(k, _):
            sl = pl.ds(k * LANE, LANE)
            sp_v[sl] = (POS_RANGE - 1) - pj_v[sl]
            sx_v[sl] = (COORD_RANGE - 1) - xj_v[sl]
            sy_v[sl] = (COORD_RANGE - 1) - yj_v[sl]
            return 0

        lax.fori_loop(0, S // LANE, shift_body, 0)

        for hh in range(PW):
            pltpu.sync_copy(w1_hbm.at[h0 + hh], w1_v)
            pltpu.sync_copy(wx_hbm.at[h0 + hh], wx_v)
            pltpu.sync_copy(wy_hbm.at[h0 + hh], wy_v)

            def fuse1_body(k, _):
                sl = pl.ds(k * LANE, LANE)
                t1_v[pl.ds(hh * LUT1_PAD + k * LANE, LANE)] = (
                    plsc.load_gather(w1_v, [lut1_v[sl]]))
                return 0

            lax.fori_loop(0, LUT1_PAD // LANE, fuse1_body, 0)

            def fuse2_body(k, _):
                sl = pl.ds(k * LANE, LANE)
                tx_v[pl.ds(hh * LUT2_PAD + k * LANE, LANE)] = (
                    plsc.load_gather(wx_v, [lut2_v[sl]]))
                ty_v[pl.ds(hh * LUT2_PAD + k * LANE, LANE)] = (
                    plsc.load_gather(wy_v, [lut2_v[sl]]))
                return 0

            lax.fori_loop(0, LUT2_PAD // LANE, fuse2_body, 0)

        t1h = [t1_v.at[pl.ds(hh * LUT1_PAD, LUT1_PAD)] for hh in range(PW)]
        txh = [tx_v.at[pl.ds(hh * LUT2_PAD, LUT2_PAD)] for hh in range(PW)]
        tyh = [ty_v.at[pl.ds(hh * LUT2_PAD, LUT2_PAD)] for hh in range(PW)]

        RBLK = 1
        CBLK = 4

        def compute_chunk(slot, i0):
            def jb_body(jb, _):
                j0 = jb * (CBLK * LANE)
                pb = [pj_v[pl.ds(j0 + t * LANE, LANE)] for t in range(CBLK)]
                xb = [xj_v[pl.ds(j0 + t * LANE, LANE)] for t in range(CBLK)]
                yb = [yj_v[pl.ds(j0 + t * LANE, LANE)] for t in range(CBLK)]

                @plsc.parallel_loop(0, ROWS, step=RBLK, unroll=4)
                def rows_body(il0):
                    for r in range(RBLK):
                        il = il0 + r
                        s1 = sp_v[pl.ds(i0 + il, LANE)][0]
                        s2 = sx_v[pl.ds(i0 + il, LANE)][0]
                        s3 = sy_v[pl.ds(i0 + il, LANE)][0]
                        for t in range(CBLK):
                            i1 = pb[t] + s1
                            i2 = xb[t] + s2
                            i3 = yb[t] + s3
                            for hh in range(PW):
                                v1 = plsc.load_gather(t1h[hh], [i1])
                                v2 = plsc.load_gather(txh[hh], [i2])
                                v3 = plsc.load_gather(tyh[hh], [i3])
                                buf_v[slot * PW + hh, il,
                                      pl.ds(j0 + t * LANE, LANE)] = (
                                    v1 + v2 + v3)

                return 0

            lax.fori_loop(0, S // (CBLK * LANE), jb_body, 0)

        def out_copies(ci, slot, sem):
            return [
                pltpu.make_async_copy(
                    buf_v.at[slot * PW + hh],
                    out_hbm.at[cplane0 + hh, pl.ds(ci * ROWS, ROWS)],
                    sem)
                for hh in range(PW)
            ]

        def pair_body(cp, _):
            ciA = cp * 2
            ciB = cp * 2 + 1

            @pl.when(cp >= 1)
            def _():
                for c in out_copies(ciA - 2, 0, semA):
                    c.wait()

            compute_chunk(0, ciA * ROWS)
            for c in out_copies(ciA, 0, semA):
                c.start()

            @pl.when(cp >= 1)
            def _():
                for c in out_copies(ciB - 2, 1, semB):
                    c.wait()

            compute_chunk(1, ciB * ROWS)
            for c in out_copies(ciB, 1, semB):
                c.start()
            return 0

        lax.fori_loop(0, NCHUNK // 2, pair_body, 0)
        for c in out_copies(NCHUNK - 2, 0, semA):
            c.wait()
        for c in out_copies(NCHUNK - 1, 1, semB):
            c.wait()

    return sc_bias


_SC_BIAS = None


def kernel(position_ids, bbox, rel_pos_bias_w, rel_pos_x_bias_w, rel_pos_y_bias_w):
    global _SC_BIAS
    if _SC_BIAS is None:
        _SC_BIAS = _sc_bias_fn()

    pos = position_ids.astype(jnp.int32)
    x = bbox[:, :, 0].astype(jnp.int32)
    y = bbox[:, :, 3].astype(jnp.int32)

    zero = jnp.minimum(pos[0, 0], 0)
    d1 = jnp.arange(-(POS_RANGE - 1), LUT1_PAD - (POS_RANGE - 1), dtype=jnp.int32) + zero
    lut1 = _bucket_ids(d1, REL_POS_BINS, MAX_REL_POS).astype(jnp.int32)
    d2 = jnp.arange(-(COORD_RANGE - 1), LUT2_PAD - (COORD_RANGE - 1), dtype=jnp.int32) + zero
    lut2 = _bucket_ids(d2, REL_2D_POS_BINS, MAX_REL_2D_POS).astype(jnp.int32)

    out = _SC_BIAS(pos, x, y, lut1, lut2,
                   rel_pos_bias_w, rel_pos_x_bias_w, rel_pos_y_bias_w)
    return out.reshape(B, H, S, S)


# speedup vs baseline: 1.0675x; 1.0675x over previous
import functools
import math

import jax
import jax.numpy as jnp
from jax import lax
from jax.experimental import pallas as pl
from jax.experimental.pallas import tpu as pltpu
from jax.experimental.pallas import tpu_sc as plsc

B, S, H = 8, 512, 12
REL_POS_BINS, MAX_REL_POS = 32, 128
REL_2D_POS_BINS, MAX_REL_2D_POS = 64, 256

POS_RANGE = 512
COORD_RANGE = 1000
LUT1_PAD = 1024
LUT2_PAD = 2048

NPLANES = B * H
NWORKERS = 32
PW = NPLANES // NWORKERS
ROWS = 32
NCHUNK = S // ROWS
LANE = 16


def _bucket_ids(delta, num_buckets, max_distance):
    nb = num_buckets // 2
    ret = (delta > 0).astype(delta.dtype) * nb
    n = jnp.abs(delta)
    max_exact = nb // 2
    is_small = n < max_exact
    val_if_large = max_exact + (
        jnp.log(n.astype(jnp.float32) / max_exact)
        / math.log(max_distance / max_exact)
        * (nb - max_exact)
    ).astype(delta.dtype)
    val_if_large = jnp.minimum(val_if_large, jnp.full_like(val_if_large, nb - 1))
    return ret + jnp.where(is_small, n, val_if_large)


def _sc_bias_fn():
    mesh = plsc.VectorSubcoreMesh(
        core_axis_name="c", subcore_axis_name="s", num_cores=2, num_subcores=16
    )

    @functools.partial(
        pl.kernel,
        out_type=jax.ShapeDtypeStruct((NPLANES, S, S), jnp.float32),
        mesh=mesh,
        compiler_params=pltpu.CompilerParams(needs_layout_passes=False),
        scratch_types=[
            pltpu.VMEM((S,), jnp.int32),
            pltpu.VMEM((S,), jnp.int32),
            pltpu.VMEM((S,), jnp.int32),
            pltpu.VMEM((S + LANE,), jnp.int32),
            pltpu.VMEM((S + LANE,), jnp.int32),
            pltpu.VMEM((S + LANE,), jnp.int32),
            pltpu.VMEM((LUT1_PAD,), jnp.int32),
            pltpu.VMEM((LUT2_PAD,), jnp.int32),
            pltpu.VMEM((REL_POS_BINS,), jnp.float32),
            pltpu.VMEM((REL_2D_POS_BINS,), jnp.float32),
            pltpu.VMEM((REL_2D_POS_BINS,), jnp.float32),
            pltpu.VMEM((PW * LUT1_PAD,), jnp.float32),
            pltpu.VMEM((PW * LUT2_PAD,), jnp.float32),
            pltpu.VMEM((PW * LUT2_PAD,), jnp.float32),
            pltpu.VMEM((2 * PW, ROWS, S), jnp.float32),
            pltpu.SemaphoreType.DMA,
            pltpu.SemaphoreType.DMA,
        ],
    )
    def sc_bias(pos_hbm, x_hbm, y_hbm, lut1_hbm, lut2_hbm,
                w1_hbm, wx_hbm, wy_hbm, out_hbm,
                pj_v, xj_v, yj_v, sp_v, sx_v, sy_v,
                lut1_v, lut2_v, w1_v, wx_v, wy_v,
                t1_v, tx_v, ty_v, buf_v, semA, semB):
        wid = lax.axis_index("s") * 2 + lax.axis_index("c")
        cplane0 = wid * PW
        b = cplane0 // H
        h0 = cplane0 - b * H

        pltpu.sync_copy(lut1_hbm, lut1_v)
        pltpu.sync_copy(lut2_hbm, lut2_v)
        pltpu.sync_copy(pos_hbm.at[b], pj_v)
        pltpu.sync_copy(x_hbm.at[b], xj_v)
        pltpu.sync_copy(y_hbm.at[b], yj_v)

        def shift_body(k, _):
            sl = pl.ds(k * LANE, LANE)
            sp_v[sl] = (POS_RANGE - 1) - pj_v[sl]
            sx_v[sl] = (COORD_RANGE - 1) - xj_v[sl]
            sy_v[sl] = (COORD_RANGE - 1) - yj_v[sl]
            return 0

        lax.fori_loop(0, S // LANE, shift_body, 0)

        for hh in range(PW):
            pltpu.sync_copy(w1_hbm.at[h0 + hh], w1_v)
            pltpu.sync_copy(wx_hbm.at[h0 + hh], wx_v)
            pltpu.sync_copy(wy_hbm.at[h0 + hh], wy_v)

            def fuse1_body(k, _):
                sl = pl.ds(k * LANE, LANE)
                t1_v[pl.ds(hh * LUT1_PAD + k * LANE, LANE)] = (
                    plsc.load_gather(w1_v, [lut1_v[sl]]))
                return 0

            lax.fori_loop(0, LUT1_PAD // LANE, fuse1_body, 0)

            def fuse2_body(k, _):
                sl = pl.ds(k * LANE, LANE)
                tx_v[pl.ds(hh * LUT2_PAD + k * LANE, LANE)] = (
                    plsc.load_gather(wx_v, [lut2_v[sl]]))
                ty_v[pl.ds(hh * LUT2_PAD + k * LANE, LANE)] = (
                    plsc.load_gather(wy_v, [lut2_v[sl]]))
                return 0

            lax.fori_loop(0, LUT2_PAD // LANE, fuse2_body, 0)

        t1h = [t1_v.at[pl.ds(hh * LUT1_PAD, LUT1_PAD)] for hh in range(PW)]
        txh = [tx_v.at[pl.ds(hh * LUT2_PAD, LUT2_PAD)] for hh in range(PW)]
        tyh = [ty_v.at[pl.ds(hh * LUT2_PAD, LUT2_PAD)] for hh in range(PW)]

        RBLK = 1
        CBLK = 4

        def compute_chunk(slot, i0):
            def jb_body(jb, _):
                j0 = jb * (CBLK * LANE)
                pb = [pj_v[pl.ds(j0 + t * LANE, LANE)] for t in range(CBLK)]
                xb = [xj_v[pl.ds(j0 + t * LANE, LANE)] for t in range(CBLK)]
                yb = [yj_v[pl.ds(j0 + t * LANE, LANE)] for t in range(CBLK)]

                @plsc.parallel_loop(0, ROWS, step=RBLK, unroll=2)
                def rows_body(il0):
                    for r in range(RBLK):
                        il = il0 + r
                        s1 = sp_v[pl.ds(i0 + il, LANE)][0]
                        s2 = sx_v[pl.ds(i0 + il, LANE)][0]
                        s3 = sy_v[pl.ds(i0 + il, LANE)][0]
                        for t in range(CBLK):
                            i1 = pb[t] + s1
                            i2 = xb[t] + s2
                            i3 = yb[t] + s3
                            for hh in range(PW):
                                v1 = plsc.load_gather(t1h[hh], [i1])
                                v2 = plsc.load_gather(txh[hh], [i2])
                                v3 = plsc.load_gather(tyh[hh], [i3])
                                buf_v[slot * PW + hh, il,
                                      pl.ds(j0 + t * LANE, LANE)] = (
                                    v1 + v2 + v3)

                return 0

            lax.fori_loop(0, S // (CBLK * LANE), jb_body, 0)

        def out_copies(ci, slot, sem):
            return [
                pltpu.make_async_copy(
                    buf_v.at[slot * PW + hh],
                    out_hbm.at[cplane0 + hh, pl.ds(ci * ROWS, ROWS)],
                    sem)
                for hh in range(PW)
            ]

        def pair_body(cp, _):
            ciA = cp * 2
            ciB = cp * 2 + 1

            @pl.when(cp >= 1)
            def _():
                for c in out_copies(ciA - 2, 0, semA):
                    c.wait()

            compute_chunk(0, ciA * ROWS)
            for c in out_copies(ciA, 0, semA):
                c.start()

            @pl.when(cp >= 1)
            def _():
                for c in out_copies(ciB - 2, 1, semB):
                    c.wait()

            compute_chunk(1, ciB * ROWS)
            for c in out_copies(ciB, 1, semB):
                c.start()
            return 0

        lax.fori_loop(0, NCHUNK // 2, pair_body, 0)
        for c in out_copies(NCHUNK - 2, 0, semA):
            c.wait()
        for c in out_copies(NCHUNK - 1, 1, semB):
            c.wait()

    return sc_bias


_SC_BIAS = None


def kernel(position_ids, bbox, rel_pos_bias_w, rel_pos_x_bias_w, rel_pos_y_bias_w):
    global _SC_BIAS
    if _SC_BIAS is None:
        _SC_BIAS = _sc_bias_fn()

    pos = position_ids.astype(jnp.int32)
    x = bbox[:, :, 0].astype(jnp.int32)
    y = bbox[:, :, 3].astype(jnp.int32)

    zero = jnp.minimum(pos[0, 0], 0)
    d1 = jnp.arange(-(POS_RANGE - 1), LUT1_PAD - (POS_RANGE - 1), dtype=jnp.int32) + zero
    lut1 = _bucket_ids(d1, REL_POS_BINS, MAX_REL_POS).astype(jnp.int32)
    d2 = jnp.arange(-(COORD_RANGE - 1), LUT2_PAD - (COORD_RANGE - 1), dtype=jnp.int32) + zero
    lut2 = _bucket_ids(d2, REL_2D_POS_BINS, MAX_REL_2D_POS).astype(jnp.int32)

    out = _SC_BIAS(pos, x, y, lut1, lut2,
                   rel_pos_bias_w, rel_pos_x_bias_w, rel_pos_y_bias_w)
    return out.reshape(B, H, S, S)


# speedup vs baseline: 1.0687x; 1.0012x over previous
import functools
import math

import jax
import jax.numpy as jnp
from jax import lax
from jax.experimental import pallas as pl
from jax.experimental.pallas import tpu as pltpu
from jax.experimental.pallas import tpu_sc as plsc

B, S, H = 8, 512, 12
REL_POS_BINS, MAX_REL_POS = 32, 128
REL_2D_POS_BINS, MAX_REL_2D_POS = 64, 256

POS_RANGE = 512
COORD_RANGE = 1000
LUT1_PAD = 1024
LUT2_PAD = 2048

NPLANES = B * H
NWORKERS = 32
PW = NPLANES // NWORKERS
ROWS = 32
NCHUNK = S // ROWS
LANE = 16


def _bucket_ids(delta, num_buckets, max_distance):
    nb = num_buckets // 2
    ret = (delta > 0).astype(delta.dtype) * nb
    n = jnp.abs(delta)
    max_exact = nb // 2
    is_small = n < max_exact
    val_if_large = max_exact + (
        jnp.log(n.astype(jnp.float32) / max_exact)
        / math.log(max_distance / max_exact)
        * (nb - max_exact)
    ).astype(delta.dtype)
    val_if_large = jnp.minimum(val_if_large, jnp.full_like(val_if_large, nb - 1))
    return ret + jnp.where(is_small, n, val_if_large)


def _sc_bias_fn():
    mesh = plsc.VectorSubcoreMesh(
        core_axis_name="c", subcore_axis_name="s", num_cores=2, num_subcores=16
    )

    @functools.partial(
        pl.kernel,
        out_type=jax.ShapeDtypeStruct((NPLANES, S, S), jnp.float32),
        mesh=mesh,
        compiler_params=pltpu.CompilerParams(
            needs_layout_passes=False, skip_device_barrier=True),
        scratch_types=[
            pltpu.VMEM((S,), jnp.int32),
            pltpu.VMEM((S,), jnp.int32),
            pltpu.VMEM((S,), jnp.int32),
            pltpu.VMEM((S + LANE,), jnp.int32),
            pltpu.VMEM((S + LANE,), jnp.int32),
            pltpu.VMEM((S + LANE,), jnp.int32),
            pltpu.VMEM((LUT1_PAD,), jnp.int32),
            pltpu.VMEM((LUT2_PAD,), jnp.int32),
            pltpu.VMEM((REL_POS_BINS,), jnp.float32),
            pltpu.VMEM((REL_2D_POS_BINS,), jnp.float32),
            pltpu.VMEM((REL_2D_POS_BINS,), jnp.float32),
            pltpu.VMEM((PW * LUT1_PAD,), jnp.float32),
            pltpu.VMEM((PW * LUT2_PAD,), jnp.float32),
            pltpu.VMEM((PW * LUT2_PAD,), jnp.float32),
            pltpu.VMEM((2 * PW, ROWS, S), jnp.float32),
            pltpu.SemaphoreType.DMA,
            pltpu.SemaphoreType.DMA,
        ],
    )
    def sc_bias(pos_hbm, x_hbm, y_hbm, lut1_hbm, lut2_hbm,
                w1_hbm, wx_hbm, wy_hbm, out_hbm,
                pj_v, xj_v, yj_v, sp_v, sx_v, sy_v,
                lut1_v, lut2_v, w1_v, wx_v, wy_v,
                t1_v, tx_v, ty_v, buf_v, semA, semB):
        wid = lax.axis_index("s") * 2 + lax.axis_index("c")
        cplane0 = wid * PW
        b = cplane0 // H
        h0 = cplane0 - b * H

        pltpu.sync_copy(lut1_hbm, lut1_v)
        pltpu.sync_copy(lut2_hbm, lut2_v)
        pltpu.sync_copy(pos_hbm.at[b], pj_v)
        pltpu.sync_copy(x_hbm.at[b], xj_v)
        pltpu.sync_copy(y_hbm.at[b], yj_v)

        def shift_body(k, _):
            sl = pl.ds(k * LANE, LANE)
            sp_v[sl] = (POS_RANGE - 1) - pj_v[sl]
            sx_v[sl] = (COORD_RANGE - 1) - xj_v[sl]
            sy_v[sl] = (COORD_RANGE - 1) - yj_v[sl]
            return 0

        lax.fori_loop(0, S // LANE, shift_body, 0)

        for hh in range(PW):
            pltpu.sync_copy(w1_hbm.at[h0 + hh], w1_v)
            pltpu.sync_copy(wx_hbm.at[h0 + hh], wx_v)
            pltpu.sync_copy(wy_hbm.at[h0 + hh], wy_v)

            def fuse1_body(k, _):
                sl = pl.ds(k * LANE, LANE)
                t1_v[pl.ds(hh * LUT1_PAD + k * LANE, LANE)] = (
                    plsc.load_gather(w1_v, [lut1_v[sl]]))
                return 0

            lax.fori_loop(0, LUT1_PAD // LANE, fuse1_body, 0)

            def fuse2_body(k, _):
                sl = pl.ds(k * LANE, LANE)
                tx_v[pl.ds(hh * LUT2_PAD + k * LANE, LANE)] = (
                    plsc.load_gather(wx_v, [lut2_v[sl]]))
                ty_v[pl.ds(hh * LUT2_PAD + k * LANE, LANE)] = (
                    plsc.load_gather(wy_v, [lut2_v[sl]]))
                return 0

            lax.fori_loop(0, LUT2_PAD // LANE, fuse2_body, 0)

        t1h = [t1_v.at[pl.ds(hh * LUT1_PAD, LUT1_PAD)] for hh in range(PW)]
        txh = [tx_v.at[pl.ds(hh * LUT2_PAD, LUT2_PAD)] for hh in range(PW)]
        tyh = [ty_v.at[pl.ds(hh * LUT2_PAD, LUT2_PAD)] for hh in range(PW)]

        RBLK = 1
        CBLK = 4

        def compute_chunk(slot, i0):
            def jb_body(jb, _):
                j0 = jb * (CBLK * LANE)
                pb = [pj_v[pl.ds(j0 + t * LANE, LANE)] for t in range(CBLK)]
                xb = [xj_v[pl.ds(j0 + t * LANE, LANE)] for t in range(CBLK)]
                yb = [yj_v[pl.ds(j0 + t * LANE, LANE)] for t in range(CBLK)]

                @plsc.parallel_loop(0, ROWS, step=RBLK, unroll=2)
                def rows_body(il0):
                    for r in range(RBLK):
                        il = il0 + r
                        s1 = sp_v[pl.ds(i0 + il, LANE)][0]
                        s2 = sx_v[pl.ds(i0 + il, LANE)][0]
                        s3 = sy_v[pl.ds(i0 + il, LANE)][0]
                        for t in range(CBLK):
                            i1 = pb[t] + s1
                            i2 = xb[t] + s2
                            i3 = yb[t] + s3
                            for hh in range(PW):
                                v1 = plsc.load_gather(t1h[hh], [i1])
                                v2 = plsc.load_gather(txh[hh], [i2])
                                v3 = plsc.load_gather(tyh[hh], [i3])
                                buf_v[slot * PW + hh, il,
                                      pl.ds(j0 + t * LANE, LANE)] = (
                                    v1 + v2 + v3)

                return 0

            lax.fori_loop(0, S // (CBLK * LANE), jb_body, 0)

        def out_copies(ci, slot, sem):
            return [
                pltpu.make_async_copy(
                    buf_v.at[slot * PW + hh],
                    out_hbm.at[cplane0 + hh, pl.ds(ci * ROWS, ROWS)],
                    sem)
                for hh in range(PW)
            ]

        def pair_body(cp, _):
            ciA = cp * 2
            ciB = cp * 2 + 1

            @pl.when(cp >= 1)
            def _():
                for c in out_copies(ciA - 2, 0, semA):
                    c.wait()

            compute_chunk(0, ciA * ROWS)
            for c in out_copies(ciA, 0, semA):
                c.start()

            @pl.when(cp >= 1)
            def _():
                for c in out_copies(ciB - 2, 1, semB):
                    c.wait()

            compute_chunk(1, ciB * ROWS)
            for c in out_copies(ciB, 1, semB):
                c.start()
            return 0

        lax.fori_loop(0, NCHUNK // 2, pair_body, 0)
        for c in out_copies(NCHUNK - 2, 0, semA):
            c.wait()
        for c in out_copies(NCHUNK - 1, 1, semB):
            c.wait()

    return sc_bias


_SC_BIAS = None


def kernel(position_ids, bbox, rel_pos_bias_w, rel_pos_x_bias_w, rel_pos_y_bias_w):
    global _SC_BIAS
    if _SC_BIAS is None:
        _SC_BIAS = _sc_bias_fn()

    pos = position_ids.astype(jnp.int32)
    x = bbox[:, :, 0].astype(jnp.int32)
    y = bbox[:, :, 3].astype(jnp.int32)

    zero = jnp.minimum(pos[0, 0], 0)
    d1 = jnp.arange(-(POS_RANGE - 1), LUT1_PAD - (POS_RANGE - 1), dtype=jnp.int32) + zero
    lut1 = _bucket_ids(d1, REL_POS_BINS, MAX_REL_POS).astype(jnp.int32)
    d2 = jnp.arange(-(COORD_RANGE - 1), LUT2_PAD - (COORD_RANGE - 1), dtype=jnp.int32) + zero
    lut2 = _bucket_ids(d2, REL_2D_POS_BINS, MAX_REL_2D_POS).astype(jnp.int32)

    out = _SC_BIAS(pos, x, y, lut1, lut2,
                   rel_pos_bias_w, rel_pos_x_bias_w, rel_pos_y_bias_w)
    return out.reshape(B, H, S, S)


# speedup vs baseline: 1.0930x; 1.0227x over previous
import functools
import math

import jax
import jax.numpy as jnp
from jax import lax
from jax.experimental import pallas as pl
from jax.experimental.pallas import tpu as pltpu
from jax.experimental.pallas import tpu_sc as plsc

B, S, H = 8, 512, 12
REL_POS_BINS, MAX_REL_POS = 32, 128
REL_2D_POS_BINS, MAX_REL_2D_POS = 64, 256

POS_RANGE = 512
COORD_RANGE = 1000
LUT1_PAD = 1024
LUT2_PAD = 2048

NPLANES = B * H
NWORKERS = 32
PW = NPLANES // NWORKERS
ROWS = 32
NCHUNK = S // ROWS
LANE = 16


def _bucket_ids(delta, num_buckets, max_distance):
    nb = num_buckets // 2
    ret = (delta > 0).astype(delta.dtype) * nb
    n = jnp.abs(delta)
    max_exact = nb // 2
    is_small = n < max_exact
    val_if_large = max_exact + (
        jnp.log(n.astype(jnp.float32) / max_exact)
        / math.log(max_distance / max_exact)
        * (nb - max_exact)
    ).astype(delta.dtype)
    val_if_large = jnp.minimum(val_if_large, jnp.full_like(val_if_large, nb - 1))
    return ret + jnp.where(is_small, n, val_if_large)


def _sc_bias_fn():
    mesh = plsc.VectorSubcoreMesh(
        core_axis_name="c", subcore_axis_name="s", num_cores=2, num_subcores=16
    )

    @functools.partial(
        pl.kernel,
        out_type=jax.ShapeDtypeStruct((NPLANES, S, S), jnp.float32),
        mesh=mesh,
        compiler_params=pltpu.CompilerParams(needs_layout_passes=False),
        scratch_types=[
            pltpu.VMEM((S,), jnp.int32),
            pltpu.VMEM((S,), jnp.int32),
            pltpu.VMEM((S,), jnp.int32),
            pltpu.VMEM((S + LANE,), jnp.int32),
            pltpu.VMEM((S + LANE,), jnp.int32),
            pltpu.VMEM((S + LANE,), jnp.int32),
            pltpu.VMEM((LUT1_PAD,), jnp.int32),
            pltpu.VMEM((LUT2_PAD,), jnp.int32),
            pltpu.VMEM((REL_POS_BINS,), jnp.float32),
            pltpu.VMEM((REL_2D_POS_BINS,), jnp.float32),
            pltpu.VMEM((REL_2D_POS_BINS,), jnp.float32),
            pltpu.VMEM((PW * LUT1_PAD,), jnp.float32),
            pltpu.VMEM((PW * LUT2_PAD,), jnp.float32),
            pltpu.VMEM((PW * LUT2_PAD,), jnp.float32),
            pltpu.VMEM((2 * PW, ROWS, S), jnp.float32),
            pltpu.SemaphoreType.DMA,
            pltpu.SemaphoreType.DMA,
        ],
    )
    def sc_bias(pos_hbm, x_hbm, y_hbm, lut1_hbm, lut2_hbm,
                w1_hbm, wx_hbm, wy_hbm, out_hbm,
                pj_v, xj_v, yj_v, sp_v, sx_v, sy_v,
                lut1_v, lut2_v, w1_v, wx_v, wy_v,
                t1_v, tx_v, ty_v, buf_v, semA, semB):
        wid = lax.axis_index("s") * 2 + lax.axis_index("c")
        cplane0 = wid * PW
        b = cplane0 // H
        h0 = cplane0 - b * H

        pltpu.sync_copy(lut1_hbm, lut1_v)
        pltpu.sync_copy(lut2_hbm, lut2_v)
        pltpu.sync_copy(pos_hbm.at[b], pj_v)
        pltpu.sync_copy(x_hbm.at[b], xj_v)
        pltpu.sync_copy(y_hbm.at[b], yj_v)

        def shift_body(k, _):
            sl = pl.ds(k * LANE, LANE)
            sp_v[sl] = (POS_RANGE - 1) - pj_v[sl]
            sx_v[sl] = (COORD_RANGE - 1) - xj_v[sl]
            sy_v[sl] = (COORD_RANGE - 1) - yj_v[sl]
            return 0

        lax.fori_loop(0, S // LANE, shift_body, 0)

        for hh in range(PW):
            pltpu.sync_copy(w1_hbm.at[h0 + hh], w1_v)
            pltpu.sync_copy(wx_hbm.at[h0 + hh], wx_v)
            pltpu.sync_copy(wy_hbm.at[h0 + hh], wy_v)

            @plsc.parallel_loop(0, LUT1_PAD // LANE, unroll=2)
            def fuse1_body(k):
                sl = pl.ds(k * LANE, LANE)
                t1_v[pl.ds(hh * LUT1_PAD + k * LANE, LANE)] = (
                    plsc.load_gather(w1_v, [lut1_v[sl]]))

            @plsc.parallel_loop(0, LUT2_PAD // LANE, unroll=2)
            def fuse2_body(k):
                sl = pl.ds(k * LANE, LANE)
                tx_v[pl.ds(hh * LUT2_PAD + k * LANE, LANE)] = (
                    plsc.load_gather(wx_v, [lut2_v[sl]]))
                ty_v[pl.ds(hh * LUT2_PAD + k * LANE, LANE)] = (
                    plsc.load_gather(wy_v, [lut2_v[sl]]))

        t1h = [t1_v.at[pl.ds(hh * LUT1_PAD, LUT1_PAD)] for hh in range(PW)]
        txh = [tx_v.at[pl.ds(hh * LUT2_PAD, LUT2_PAD)] for hh in range(PW)]
        tyh = [ty_v.at[pl.ds(hh * LUT2_PAD, LUT2_PAD)] for hh in range(PW)]

        RBLK = 1
        CBLK = 4

        def compute_chunk(slot, i0):
            def jb_body(jb, _):
                j0 = jb * (CBLK * LANE)
                pb = [pj_v[pl.ds(j0 + t * LANE, LANE)] for t in range(CBLK)]
                xb = [xj_v[pl.ds(j0 + t * LANE, LANE)] for t in range(CBLK)]
                yb = [yj_v[pl.ds(j0 + t * LANE, LANE)] for t in range(CBLK)]

                @plsc.parallel_loop(0, ROWS, step=RBLK, unroll=2)
                def rows_body(il0):
                    for r in range(RBLK):
                        il = il0 + r
                        s1 = sp_v[pl.ds(i0 + il, LANE)][0]
                        s2 = sx_v[pl.ds(i0 + il, LANE)][0]
                        s3 = sy_v[pl.ds(i0 + il, LANE)][0]
                        for t in range(CBLK):
                            i1 = pb[t] + s1
                            i2 = xb[t] + s2
                            i3 = yb[t] + s3
                            for hh in range(PW):
                                v1 = plsc.load_gather(t1h[hh], [i1])
                                v2 = plsc.load_gather(txh[hh], [i2])
                                v3 = plsc.load_gather(tyh[hh], [i3])
                                buf_v[slot * PW + hh, il,
                                      pl.ds(j0 + t * LANE, LANE)] = (
                                    v1 + v2 + v3)

                return 0

            lax.fori_loop(0, S // (CBLK * LANE), jb_body, 0)

        def out_copies(ci, slot, sem):
            return [
                pltpu.make_async_copy(
                    buf_v.at[slot * PW + hh],
                    out_hbm.at[cplane0 + hh, pl.ds(ci * ROWS, ROWS)],
                    sem)
                for hh in range(PW)
            ]

        def pair_body(cp, _):
            ciA = cp * 2
            ciB = cp * 2 + 1

            @pl.when(cp >= 1)
            def _():
                for c in out_copies(ciA - 2, 0, semA):
                    c.wait()

            compute_chunk(0, ciA * ROWS)
            for c in out_copies(ciA, 0, semA):
                c.start()

            @pl.when(cp >= 1)
            def _():
                for c in out_copies(ciB - 2, 1, semB):
                    c.wait()

            compute_chunk(1, ciB * ROWS)
            for c in out_copies(ciB, 1, semB):
                c.start()
            return 0

        lax.fori_loop(0, NCHUNK // 2, pair_body, 0)
        for c in out_copies(NCHUNK - 2, 0, semA):
            c.wait()
        for c in out_copies(NCHUNK - 1, 1, semB):
            c.wait()

    return sc_bias


_SC_BIAS = None


def kernel(position_ids, bbox, rel_pos_bias_w, rel_pos_x_bias_w, rel_pos_y_bias_w):
    global _SC_BIAS
    if _SC_BIAS is None:
        _SC_BIAS = _sc_bias_fn()

    pos = position_ids.astype(jnp.int32)
    x = bbox[:, :, 0].astype(jnp.int32)
    y = bbox[:, :, 3].astype(jnp.int32)

    zero = jnp.minimum(pos[0, 0], 0)
    d1 = jnp.arange(-(POS_RANGE - 1), LUT1_PAD - (POS_RANGE - 1), dtype=jnp.int32) + zero
    lut1 = _bucket_ids(d1, REL_POS_BINS, MAX_REL_POS).astype(jnp.int32)
    d2 = jnp.arange(-(COORD_RANGE - 1), LUT2_PAD - (COORD_RANGE - 1), dtype=jnp.int32) + zero
    lut2 = _bucket_ids(d2, REL_2D_POS_BINS, MAX_REL_2D_POS).astype(jnp.int32)

    out = _SC_BIAS(pos, x, y, lut1, lut2,
                   rel_pos_bias_w, rel_pos_x_bias_w, rel_pos_y_bias_w)
    return out.reshape(B, H, S, S)
